# baseline (speedup 1.0000x reference)
import functools

import jax
import jax.numpy as jnp
import numpy as np
from jax import lax
from jax.experimental import pallas as pl
from jax.experimental.pallas import tpu as pltpu
from jax.experimental.pallas import tpu_sc as plsc

B, C, H, W = 4, 512, 32, 32
N = H * W
K = 16
RT = 128
NT = N // RT
BN = B * N

NW = 32
RPW = BN // NW
RB = 16
NCH = N // 16

_NEG_INF = np.float32(-np.inf)


def _sims_kernel(xv_ref, xi_ref, sv_ref, si_ref, nv_ref, ni_ref):
    t = pl.program_id(1)

    @pl.when(t == 0)
    def _():
        xv = xv_ref[0]
        xi = xi_ref[0]
        nv_ref[...] = xv * (1.0 / (jnp.sqrt(jnp.sum(xv * xv, axis=1,
                                                    keepdims=True)) + 1e-6))
        ni_ref[...] = xi * (1.0 / (jnp.sqrt(jnp.sum(xi * xi, axis=1,
                                                    keepdims=True)) + 1e-6))

    rows_v = nv_ref[pl.ds(t * RT, RT), :]
    rows_i = ni_ref[pl.ds(t * RT, RT), :]

    dn = (((1,), (1,)), ((), ()))
    sv_ref[...] = jax.lax.dot_general(rows_v, nv_ref[...], dn,
                                      preferred_element_type=jnp.float32)
    si_ref[...] = jax.lax.dot_general(rows_i, ni_ref[...], dn,
                                      preferred_element_type=jnp.float32)


def _sc_topk_kernel(rpw, sv_hbm, si_hbm, cv_hbm, ci_hbm, bufv, bufi, ovb,
                    oib, semv, semi):
    cid = lax.axis_index("c")
    sid = lax.axis_index("s")
    wid = sid * 2 + cid
    base = wid * rpw
    nblk = rpw // RB

    lane = lax.iota(jnp.int32, 16)

    def _start(blk):
        st = base + blk * RB
        pltpu.async_copy(sv_hbm.at[pl.ds(st, RB), :], bufv.at[blk % 2],
                         semv.at[blk % 2])
        pltpu.async_copy(si_hbm.at[pl.ds(st, RB), :], bufi.at[blk % 2],
                         semi.at[blk % 2])

    for blk in range(min(2, nblk)):
        _start(blk)

    for blk in range(nblk):
        start = base + blk * RB
        pltpu.make_async_copy(sv_hbm.at[pl.ds(start, RB), :],
                              bufv.at[blk % 2], semv.at[blk % 2]).wait()
        pltpu.make_async_copy(si_hbm.at[pl.ds(start, RB), :],
                              bufi.at[blk % 2], semi.at[blk % 2]).wait()
        bv = bufv.at[blk % 2]
        bi = bufi.at[blk % 2]

        def one_row(r, _, start=start, bv=bv, bi=bi, blk=blk):
            n_glob = start + r
            n_self = lax.rem(n_glob, N)
            jc = lax.div(n_self, 16) * 16
            ln = lax.rem(n_self, 16)
            sv_chunk = bv[r, pl.ds(jc, 16)]
            bv[r, pl.ds(jc, 16)] = jnp.where(lane == ln, _NEG_INF, sv_chunk)

            def mrg(asc, desc, to_desc):
                take = desc[0] > asc[0]
                k = jnp.where(take, desc[0], asc[0])
                i = jnp.where(take, desc[1], asc[1])
                return tuple(plsc.sort_key_val(k, i, descending=to_desc))

            def ldkey(j):
                return bv[r, pl.ds(j * 16, 16)], lane + j * 16

            def group(g, carry):
                j0 = g * 4
                k1, i1 = ldkey(j0)
                k2, i2 = ldkey(j0 + 1)
                k3, i3 = ldkey(j0 + 2)
                k4, i4 = ldkey(j0 + 3)
                s1 = tuple(plsc.sort_key_val(k1, i1, descending=False))
                s2 = tuple(plsc.sort_key_val(k2, i2, descending=True))
                s3 = tuple(plsc.sort_key_val(k3, i3, descending=False))
                s4 = tuple(plsc.sort_key_val(k4, i4, descending=True))
                d12 = mrg(s1, s2, True)
                a34 = mrg(s3, s4, False)
                gtop = mrg(a34, d12, True)
                return mrg(carry, gtop, False)

            init = (jnp.full((16,), _NEG_INF, jnp.float32),
                    jnp.zeros((16,), jnp.int32))
            rk, ri = lax.fori_loop(0, NCH // 4, group, init)

            idx_d = lax.rev(ri, (0,))
            rvec = jnp.zeros((16,), jnp.int32) + r
            cv = plsc.load_gather(bv, [rvec, idx_d])
            cvi = plsc.load_gather(bi, [rvec, idx_d])
            ovb[blk * RB + r, :] = cv
            oib[blk * RB + r, :] = cvi
            return 0

        lax.fori_loop(0, RB, one_row, 0)
        if blk + 2 < nblk:
            _start(blk + 2)

    pltpu.sync_copy(ovb, cv_hbm.at[pl.ds(base, rpw), :])
    pltpu.sync_copy(oib, ci_hbm.at[pl.ds(base, rpw), :])


def _sc_topk(simv, simi):
    rows = simv.shape[0]
    rpw = rows // NW
    mesh = plsc.VectorSubcoreMesh(core_axis_name="c", subcore_axis_name="s")
    f = functools.partial(
        pl.kernel,
        mesh=mesh,
        out_type=[jax.ShapeDtypeStruct((rows, K), jnp.float32),
                  jax.ShapeDtypeStruct((rows, K), jnp.float32)],
        scratch_types=[pltpu.VMEM((2, RB, N), jnp.float32),
                       pltpu.VMEM((2, RB, N), jnp.float32),
                       pltpu.VMEM((rpw, K), jnp.float32),
                       pltpu.VMEM((rpw, K), jnp.float32),
                       pltpu.SemaphoreType.DMA((2,)),
                       pltpu.SemaphoreType.DMA((2,))],
        compiler_params=pltpu.CompilerParams(needs_layout_passes=False),
    )(functools.partial(_sc_topk_kernel, rpw))
    return f(simv, simi)


def _mlp_mask(Cmat, W1, b1, W2, b2_scalar):
    h = jax.lax.dot_general(Cmat, W1, (((1,), (1,)), ((), ())),
                            preferred_element_type=jnp.float32)
    h = jnp.maximum(h + b1, 0.0)
    z = jnp.sum(h * W2, axis=1, keepdims=True) + b2_scalar
    return 1.0 / (1.0 + jnp.exp(-z))


def _post_kernel(xv_ref, xi_ref, cv_ref, ci_ref, w1_ref, b1_ref, w2_ref,
                 b2_ref, t_ref, ov_ref, oi_ref, loss_ref):
    g = pl.program_id(0)
    inv_t = 1.0 / t_ref[0]

    Cv = cv_ref[...] * inv_t
    Ci = ci_ref[...] * inv_t

    b2s = b2_ref[0]
    mv = _mlp_mask(Cv, w1_ref[...], b1_ref[...], w2_ref[...], b2s)
    mi = _mlp_mask(Ci, w1_ref[...], b1_ref[...], w2_ref[...], b2s)

    ov_ref[...] = xv_ref[...] * mv
    oi_ref[...] = xi_ref[...] * mi

    eps = np.float32(1e-12)
    cvn = Cv / jnp.maximum(jnp.sqrt(jnp.sum(Cv * Cv, axis=1, keepdims=True)), eps)
    cin = Ci / jnp.maximum(jnp.sqrt(jnp.sum(Ci * Ci, axis=1, keepdims=True)), eps)
    part = jnp.sum((cvn - cin) ** 2).reshape(1, 1)

    @pl.when(g == 0)
    def _():
        loss_ref[:, :] = jnp.zeros((1, 1), jnp.float32)

    loss_ref[:, :] += part


NSPLIT = 4
IPS = B // NSPLIT


def _stage_a(Fv_h, Fi_h):
    rows_h = IPS * N
    full = pl.BlockSpec((1, N, C), lambda bb, tt: (bb, 0, 0))
    simb = pl.BlockSpec((RT, N), lambda bb, tt: (bb * NT + tt, 0))
    return pl.pallas_call(
        _sims_kernel,
        grid=(IPS, NT),
        in_specs=[full, full],
        out_specs=[simb, simb],
        out_shape=[jax.ShapeDtypeStruct((rows_h, N), jnp.float32),
                   jax.ShapeDtypeStruct((rows_h, N), jnp.float32)],
        scratch_shapes=[pltpu.VMEM((N, C), jnp.float32),
                        pltpu.VMEM((N, C), jnp.float32)],
    )(Fv_h, Fi_h)


def _stage_b(Fv2, Fi2, cv, ci, W1, b1, W2, b2, temperature):
    rows_h = Fv2.shape[0]
    rows = pl.BlockSpec((RT, C), lambda g: (g, 0))
    cb = pl.BlockSpec((RT, K), lambda g: (g, 0))
    rep = lambda shape: pl.BlockSpec(shape, lambda g: tuple(0 for _ in shape))
    return pl.pallas_call(
        _post_kernel,
        grid=(rows_h // RT,),
        in_specs=[rows, rows, cb, cb,
                  rep((32, K)), rep((1, 32)), rep((1, 32)),
                  pl.BlockSpec(memory_space=pltpu.SMEM),
                  pl.BlockSpec(memory_space=pltpu.SMEM)],
        out_specs=[rows, rows, pl.BlockSpec((1, 1), lambda g: (0, 0))],
        out_shape=[jax.ShapeDtypeStruct((rows_h, C), jnp.float32),
                   jax.ShapeDtypeStruct((rows_h, C), jnp.float32),
                   jax.ShapeDtypeStruct((1, 1), jnp.float32)],
    )(Fv2, Fi2, cv, ci, W1, b1.reshape(1, 32), W2, b2.reshape(1),
      temperature.reshape(1))


@jax.jit
def kernel(F_v, F_i, W1, b1, W2, b2, temperature):
    b, c, h, w = F_v.shape
    Fv = F_v.reshape(b, c, h * w).transpose(0, 2, 1)
    Fi = F_i.reshape(b, c, h * w).transpose(0, 2, 1)

    Fhs, sims, cs = [], [], []
    for sidx in range(NSPLIT):
        Fv_h = lax.slice_in_dim(Fv, sidx * IPS, (sidx + 1) * IPS, axis=0)
        Fi_h = lax.slice_in_dim(Fi, sidx * IPS, (sidx + 1) * IPS, axis=0)
        Fhs.append((Fv_h, Fi_h))
        sims.append(_stage_a(Fv_h, Fi_h))
    for sidx in range(NSPLIT):
        cs.append(_sc_topk(*sims[sidx]))
    ovs, ois, losses = [], [], []
    for sidx in range(NSPLIT):
        Fv_h, Fi_h = Fhs[sidx]
        cv, ci = cs[sidx]
        ov, oi, loss = _stage_b(Fv_h.reshape(IPS * N, C),
                                Fi_h.reshape(IPS * N, C), cv, ci,
                                W1, b1, W2, b2, temperature)
        ovs.append(ov.reshape(IPS, N, C))
        ois.append(oi.reshape(IPS, N, C))
        losses.append(loss[0, 0])

    ov = jnp.concatenate(ovs, axis=0)
    oi = jnp.concatenate(ois, axis=0)
    F_v_den = ov.transpose(0, 2, 1).reshape(b, c, h, w)
    F_i_den = oi.transpose(0, 2, 1).reshape(b, c, h, w)
    loss_syn = (sum(losses) * (100.0 / (B * N * K))).astype(jnp.float32)
    return (F_v_den, F_i_den, loss_syn)


# speedup vs baseline: 1.0660x; 1.0660x over previous
import functools

import jax
import jax.numpy as jnp
import numpy as np
from jax import lax
from jax.experimental import pallas as pl
from jax.experimental.pallas import tpu as pltpu
from jax.experimental.pallas import tpu_sc as plsc

B, C, H, W = 4, 512, 32, 32
N = H * W
K = 16
RT = 128
NT = N // RT
BN = B * N

NW = 32
RPW = BN // NW
RB = 8
NCH = N // 16

_NEG_INF = np.float32(-np.inf)


def _sims_kernel(xv_ref, xi_ref, sv_ref, si_ref, nv_ref, ni_ref):
    t = pl.program_id(1)

    @pl.when(t == 0)
    def _():
        xv = xv_ref[0]
        xi = xi_ref[0]
        nv_ref[...] = xv * (1.0 / (jnp.sqrt(jnp.sum(xv * xv, axis=1,
                                                    keepdims=True)) + 1e-6))
        ni_ref[...] = xi * (1.0 / (jnp.sqrt(jnp.sum(xi * xi, axis=1,
                                                    keepdims=True)) + 1e-6))

    rows_v = nv_ref[pl.ds(t * RT, RT), :]
    rows_i = ni_ref[pl.ds(t * RT, RT), :]

    dn = (((1,), (1,)), ((), ()))
    sv_ref[...] = jax.lax.dot_general(rows_v, nv_ref[...], dn,
                                      preferred_element_type=jnp.float32)
    si_ref[...] = jax.lax.dot_general(rows_i, ni_ref[...], dn,
                                      preferred_element_type=jnp.float32)


def _sc_topk_kernel(rpw, sv_hbm, si_hbm, cv_hbm, ci_hbm, bufv, bufi, ovb,
                    oib, semv, semi):
    cid = lax.axis_index("c")
    sid = lax.axis_index("s")
    wid = sid * 2 + cid
    base = wid * rpw
    nblk = rpw // RB

    lane = lax.iota(jnp.int32, 16)

    def _start(blk):
        st = base + blk * RB
        pltpu.async_copy(sv_hbm.at[pl.ds(st, RB), :], bufv.at[blk % 2],
                         semv.at[blk % 2])
        pltpu.async_copy(si_hbm.at[pl.ds(st, RB), :], bufi.at[blk % 2],
                         semi.at[blk % 2])

    for blk in range(min(2, nblk)):
        _start(blk)

    for blk in range(nblk):
        start = base + blk * RB
        pltpu.make_async_copy(sv_hbm.at[pl.ds(start, RB), :],
                              bufv.at[blk % 2], semv.at[blk % 2]).wait()
        pltpu.make_async_copy(si_hbm.at[pl.ds(start, RB), :],
                              bufi.at[blk % 2], semi.at[blk % 2]).wait()
        bv = bufv.at[blk % 2]
        bi = bufi.at[blk % 2]

        def one_row(r, _, start=start, bv=bv, bi=bi, blk=blk):
            n_glob = start + r
            n_self = lax.rem(n_glob, N)
            jc = lax.div(n_self, 16) * 16
            ln = lax.rem(n_self, 16)
            sv_chunk = bv[r, pl.ds(jc, 16)]
            bv[r, pl.ds(jc, 16)] = jnp.where(lane == ln, _NEG_INF, sv_chunk)

            def mrg(asc, desc, to_desc):
                take = desc[0] > asc[0]
                k = jnp.where(take, desc[0], asc[0])
                i = jnp.where(take, desc[1], asc[1])
                return tuple(plsc.sort_key_val(k, i, descending=to_desc))

            def ldkey(j):
                return bv[r, pl.ds(j * 16, 16)], lane + j * 16

            def group(g, carry):
                j0 = g * 4
                k1, i1 = ldkey(j0)
                k2, i2 = ldkey(j0 + 1)
                k3, i3 = ldkey(j0 + 2)
                k4, i4 = ldkey(j0 + 3)
                s1 = tuple(plsc.sort_key_val(k1, i1, descending=False))
                s2 = tuple(plsc.sort_key_val(k2, i2, descending=True))
                s3 = tuple(plsc.sort_key_val(k3, i3, descending=False))
                s4 = tuple(plsc.sort_key_val(k4, i4, descending=True))
                d12 = mrg(s1, s2, True)
                a34 = mrg(s3, s4, False)
                gtop = mrg(a34, d12, True)
                return mrg(carry, gtop, False)

            init = (jnp.full((16,), _NEG_INF, jnp.float32),
                    jnp.zeros((16,), jnp.int32))
            rk, ri = lax.fori_loop(0, NCH // 4, group, init)

            idx_d = lax.rev(ri, (0,))
            rvec = jnp.zeros((16,), jnp.int32) + r
            cv = plsc.load_gather(bv, [rvec, idx_d])
            cvi = plsc.load_gather(bi, [rvec, idx_d])
            ovb[blk * RB + r, :] = cv
            oib[blk * RB + r, :] = cvi
            return 0

        lax.fori_loop(0, RB, one_row, 0)
        if blk + 2 < nblk:
            _start(blk + 2)

    pltpu.sync_copy(ovb, cv_hbm.at[pl.ds(base, rpw), :])
    pltpu.sync_copy(oib, ci_hbm.at[pl.ds(base, rpw), :])


def _sc_topk(simv, simi):
    rows = simv.shape[0]
    rpw = rows // NW
    mesh = plsc.VectorSubcoreMesh(core_axis_name="c", subcore_axis_name="s")
    f = functools.partial(
        pl.kernel,
        mesh=mesh,
        out_type=[jax.ShapeDtypeStruct((rows, K), jnp.float32),
                  jax.ShapeDtypeStruct((rows, K), jnp.float32)],
        scratch_types=[pltpu.VMEM((2, RB, N), jnp.float32),
                       pltpu.VMEM((2, RB, N), jnp.float32),
                       pltpu.VMEM((rpw, K), jnp.float32),
                       pltpu.VMEM((rpw, K), jnp.float32),
                       pltpu.SemaphoreType.DMA((2,)),
                       pltpu.SemaphoreType.DMA((2,))],
        compiler_params=pltpu.CompilerParams(needs_layout_passes=False),
    )(functools.partial(_sc_topk_kernel, rpw))
    return f(simv, simi)


def _mlp_mask(Cmat, W1, b1, W2, b2_scalar):
    h = jax.lax.dot_general(Cmat, W1, (((1,), (1,)), ((), ())),
                            preferred_element_type=jnp.float32)
    h = jnp.maximum(h + b1, 0.0)
    z = jnp.sum(h * W2, axis=1, keepdims=True) + b2_scalar
    return 1.0 / (1.0 + jnp.exp(-z))


def _post_kernel(xv_ref, xi_ref, cv_ref, ci_ref, w1_ref, b1_ref, w2_ref,
                 b2_ref, t_ref, ov_ref, oi_ref, loss_ref):
    g = pl.program_id(0)
    inv_t = 1.0 / t_ref[0]

    Cv = cv_ref[...] * inv_t
    Ci = ci_ref[...] * inv_t

    b2s = b2_ref[0]
    mv = _mlp_mask(Cv, w1_ref[...], b1_ref[...], w2_ref[...], b2s)
    mi = _mlp_mask(Ci, w1_ref[...], b1_ref[...], w2_ref[...], b2s)

    ov_ref[...] = xv_ref[...] * mv
    oi_ref[...] = xi_ref[...] * mi

    eps = np.float32(1e-12)
    cvn = Cv / jnp.maximum(jnp.sqrt(jnp.sum(Cv * Cv, axis=1, keepdims=True)), eps)
    cin = Ci / jnp.maximum(jnp.sqrt(jnp.sum(Ci * Ci, axis=1, keepdims=True)), eps)
    part = jnp.sum((cvn - cin) ** 2).reshape(1, 1)

    @pl.when(g == 0)
    def _():
        loss_ref[:, :] = jnp.zeros((1, 1), jnp.float32)

    loss_ref[:, :] += part


NSPLIT = 2
IPS = B // NSPLIT


def _stage_a(Fv_h, Fi_h):
    rows_h = IPS * N
    full = pl.BlockSpec((1, N, C), lambda bb, tt: (bb, 0, 0))
    simb = pl.BlockSpec((RT, N), lambda bb, tt: (bb * NT + tt, 0))
    return pl.pallas_call(
        _sims_kernel,
        grid=(IPS, NT),
        in_specs=[full, full],
        out_specs=[simb, simb],
        out_shape=[jax.ShapeDtypeStruct((rows_h, N), jnp.float32),
                   jax.ShapeDtypeStruct((rows_h, N), jnp.float32)],
        scratch_shapes=[pltpu.VMEM((N, C), jnp.float32),
                        pltpu.VMEM((N, C), jnp.float32)],
    )(Fv_h, Fi_h)


def _stage_b(Fv2, Fi2, cv, ci, W1, b1, W2, b2, temperature):
    rows_h = Fv2.shape[0]
    rows = pl.BlockSpec((RT, C), lambda g: (g, 0))
    cb = pl.BlockSpec((RT, K), lambda g: (g, 0))
    rep = lambda shape: pl.BlockSpec(shape, lambda g: tuple(0 for _ in shape))
    return pl.pallas_call(
        _post_kernel,
        grid=(rows_h // RT,),
        in_specs=[rows, rows, cb, cb,
                  rep((32, K)), rep((1, 32)), rep((1, 32)),
                  pl.BlockSpec(memory_space=pltpu.SMEM),
                  pl.BlockSpec(memory_space=pltpu.SMEM)],
        out_specs=[rows, rows, pl.BlockSpec((1, 1), lambda g: (0, 0))],
        out_shape=[jax.ShapeDtypeStruct((rows_h, C), jnp.float32),
                   jax.ShapeDtypeStruct((rows_h, C), jnp.float32),
                   jax.ShapeDtypeStruct((1, 1), jnp.float32)],
    )(Fv2, Fi2, cv, ci, W1, b1.reshape(1, 32), W2, b2.reshape(1),
      temperature.reshape(1))


@jax.jit
def kernel(F_v, F_i, W1, b1, W2, b2, temperature):
    b, c, h, w = F_v.shape
    Fv = F_v.reshape(b, c, h * w).transpose(0, 2, 1)
    Fi = F_i.reshape(b, c, h * w).transpose(0, 2, 1)

    Fhs, sims, cs = [], [], []
    for sidx in range(NSPLIT):
        Fv_h = lax.slice_in_dim(Fv, sidx * IPS, (sidx + 1) * IPS, axis=0)
        Fi_h = lax.slice_in_dim(Fi, sidx * IPS, (sidx + 1) * IPS, axis=0)
        Fhs.append((Fv_h, Fi_h))
        sims.append(_stage_a(Fv_h, Fi_h))
    for sidx in range(NSPLIT):
        cs.append(_sc_topk(*sims[sidx]))
    ovs, ois, losses = [], [], []
    for sidx in range(NSPLIT):
        Fv_h, Fi_h = Fhs[sidx]
        cv, ci = cs[sidx]
        ov, oi, loss = _stage_b(Fv_h.reshape(IPS * N, C),
                                Fi_h.reshape(IPS * N, C), cv, ci,
                                W1, b1, W2, b2, temperature)
        ovs.append(ov.reshape(IPS, N, C))
        ois.append(oi.reshape(IPS, N, C))
        losses.append(loss[0, 0])

    ov = jnp.concatenate(ovs, axis=0)
    oi = jnp.concatenate(ois, axis=0)
    F_v_den = ov.transpose(0, 2, 1).reshape(b, c, h, w)
    F_i_den = oi.transpose(0, 2, 1).reshape(b, c, h, w)
    loss_syn = (sum(losses) * (100.0 / (B * N * K))).astype(jnp.float32)
    return (F_v_den, F_i_den, loss_syn)
